# baseline (speedup 1.0000x reference)
import functools

import jax
import jax.numpy as jnp
from jax import lax
from jax.experimental import pallas as pl
from jax.experimental.pallas import tpu as pltpu
from jax.experimental.pallas import tpu_sc as plsc

NC = 2
NS = 16
NW = NC * NS
LANES = 16
D = 1024
CHUNK = 16
NSLOT = 3


def _make_embed(batch, seqlen):
    b_per_w = batch * seqlen // NW
    n_chunks = b_per_w // CHUNK
    w_per_batch = NW // batch
    mesh = plsc.VectorSubcoreMesh(core_axis_name="c", subcore_axis_name="s")

    @functools.partial(
        pl.kernel,
        mesh=mesh,
        out_type=jax.ShapeDtypeStruct((batch, seqlen, D), jnp.float32),
        scratch_types=(
            [pltpu.VMEM((b_per_w,), jnp.int32)] * 2
            + [pltpu.VMEM((CHUNK, D), jnp.float32) for _ in range(2 * NSLOT)]
            + [pltpu.SemaphoreType.DMA for _ in range(3 * NSLOT)]
        ),
    )
    def k(ids_hbm, pids_hbm, wtab_hbm, ptab_hbm, out_hbm,
          idx_v, pidx_v, *bufs_sems):
        wbufs = bufs_sems[0:NSLOT]
        pbufs = bufs_sems[NSLOT:2 * NSLOT]
        sems_w = bufs_sems[2 * NSLOT:3 * NSLOT]
        sems_p = bufs_sems[3 * NSLOT:4 * NSLOT]
        sems_o = bufs_sems[4 * NSLOT:5 * NSLOT]

        wid = lax.axis_index("s") * NC + lax.axis_index("c")
        b = wid // w_per_batch
        col0 = (wid % w_per_batch) * b_per_w
        hi = pltpu.async_copy(ids_hbm.at[b, pl.ds(col0, b_per_w)], idx_v,
                              sems_w[0])
        hpi = pltpu.async_copy(pids_hbm.at[b, pl.ds(col0, b_per_w)], pidx_v,
                               sems_p[0])
        hi.wait()
        hpi.wait()

        def start_in(c):
            s = c % NSLOT
            hw = pltpu.async_copy(
                wtab_hbm.at[idx_v.at[pl.ds(c * CHUNK, CHUNK)]],
                wbufs[s], sems_w[s])
            hp = pltpu.async_copy(
                ptab_hbm.at[pidx_v.at[pl.ds(c * CHUNK, CHUNK)]],
                pbufs[s], sems_p[s])
            return hw, hp

        def add_rows(s, lo, hi):
            wbuf, pbuf = wbufs[s], pbufs[s]
            half = D // 2

            def half_row_body(h, _):
                r = h // 2
                cb = (h % 2) * half
                for j in range(half // LANES):
                    sl = pl.ds(cb + j * LANES, LANES)
                    wbuf[r, sl] = wbuf[r, sl] + pbuf[r, sl]
                return 0

            lax.fori_loop(2 * lo, 2 * hi, half_row_body, 0)

        in_flight = {}
        out_flight = {}
        for c in range(min(2, n_chunks)):
            in_flight[c] = start_in(c)

        for c in range(n_chunks):
            s = c % NSLOT
            hw, hp = in_flight.pop(c)
            hw.wait()
            hp.wait()
            add_rows(s, 0, CHUNK // 2)
            if (c - 1) in out_flight:
                out_flight.pop(c - 1).wait()
            if c + 2 < n_chunks:
                in_flight[c + 2] = start_in(c + 2)
            add_rows(s, CHUNK // 2, CHUNK)
            out_flight[c] = pltpu.async_copy(
                wbufs[s],
                out_hbm.at[b, pl.ds(col0 + c * CHUNK, CHUNK)],
                sems_o[s])
        for h in out_flight.values():
            h.wait()

    return k


def kernel(input_ids, position_ids, word_embeddings, position_embeddings):
    batch, seqlen = input_ids.shape
    embed = _make_embed(batch, seqlen)
    return embed(input_ids.astype(jnp.int32), position_ids.astype(jnp.int32),
                 word_embeddings, position_embeddings)


# speedup vs baseline: 1.1344x; 1.1344x over previous
import functools

import jax
import jax.numpy as jnp
from jax import lax
from jax.experimental import pallas as pl
from jax.experimental.pallas import tpu as pltpu
from jax.experimental.pallas import tpu_sc as plsc

NC = 2
NS = 16
NW = NC * NS
LANES = 16
D = 1024
CHUNK = 16
NSLOT = 3


def _make_embed(batch, seqlen):
    b_per_w = batch * seqlen // NW
    n_chunks = b_per_w // CHUNK
    w_per_batch = NW // batch
    mesh = plsc.VectorSubcoreMesh(core_axis_name="c", subcore_axis_name="s")

    @functools.partial(
        pl.kernel,
        mesh=mesh,
        out_type=jax.ShapeDtypeStruct((batch, seqlen, D), jnp.float32),
        scratch_types=(
            [pltpu.VMEM((b_per_w,), jnp.int32)] * 2
            + [pltpu.VMEM((CHUNK, D), jnp.float32) for _ in range(2 * NSLOT)]
            + [pltpu.SemaphoreType.DMA for _ in range(3 * NSLOT)]
        ),
    )
    def k(ids_hbm, pids_hbm, wtab_hbm, ptab_hbm, out_hbm,
          idx_v, pidx_v, *bufs_sems):
        wbufs = bufs_sems[0:NSLOT]
        pbufs = bufs_sems[NSLOT:2 * NSLOT]
        sems_w = bufs_sems[2 * NSLOT:3 * NSLOT]
        sems_p = bufs_sems[3 * NSLOT:4 * NSLOT]
        sems_o = bufs_sems[4 * NSLOT:5 * NSLOT]

        wid = lax.axis_index("s") * NC + lax.axis_index("c")
        b = wid // w_per_batch
        col0 = (wid % w_per_batch) * b_per_w
        hi = pltpu.async_copy(ids_hbm.at[b, pl.ds(col0, b_per_w)], idx_v,
                              sems_w[0])
        hpi = pltpu.async_copy(pids_hbm.at[b, pl.ds(col0, b_per_w)], pidx_v,
                               sems_p[0])
        hi.wait()
        hpi.wait()

        def start_in(c):
            s = c % NSLOT
            hw = pltpu.async_copy(
                wtab_hbm.at[idx_v.at[pl.ds(c * CHUNK, CHUNK)]],
                wbufs[s], sems_w[s])
            hp = pltpu.async_copy(
                ptab_hbm.at[pidx_v.at[pl.ds(c * CHUNK, CHUNK)]],
                pbufs[s], sems_p[s])
            return hw, hp

        def add_chunk(s):
            wbuf, pbuf = wbufs[s], pbufs[s]

            @plsc.parallel_loop(0, CHUNK)
            def row_body(r):
                for j in range(D // LANES):
                    sl = pl.ds(j * LANES, LANES)
                    wbuf[r, sl] = wbuf[r, sl] + pbuf[r, sl]

        in_flight = {}
        out_flight = {}
        for c in range(min(2, n_chunks)):
            in_flight[c] = start_in(c)

        for c in range(n_chunks):
            s = c % NSLOT
            hw, hp = in_flight.pop(c)
            hw.wait()
            hp.wait()
            add_chunk(s)
            if (c - 1) in out_flight:
                out_flight.pop(c - 1).wait()
            out_flight[c] = pltpu.async_copy(
                wbufs[s],
                out_hbm.at[b, pl.ds(col0 + c * CHUNK, CHUNK)],
                sems_o[s])
            if c + 2 < n_chunks:
                in_flight[c + 2] = start_in(c + 2)
        for h in out_flight.values():
            h.wait()

    return k


def kernel(input_ids, position_ids, word_embeddings, position_embeddings):
    batch, seqlen = input_ids.shape
    embed = _make_embed(batch, seqlen)
    return embed(input_ids.astype(jnp.int32), position_ids.astype(jnp.int32),
                 word_embeddings, position_embeddings)


# speedup vs baseline: 1.1674x; 1.0291x over previous
import functools

import jax
import jax.numpy as jnp
from jax import lax
from jax.experimental import pallas as pl
from jax.experimental.pallas import tpu as pltpu
from jax.experimental.pallas import tpu_sc as plsc

NC = 2
NS = 16
NW = NC * NS
LANES = 16
D = 1024
CHUNK = 16
NW_SLOT = 4
NP_SLOT = 2


def _make_embed(batch, seqlen):
    b_per_w = batch * seqlen // NW
    n_chunks = b_per_w // CHUNK
    w_per_batch = NW // batch
    mesh = plsc.VectorSubcoreMesh(core_axis_name="c", subcore_axis_name="s")

    @functools.partial(
        pl.kernel,
        mesh=mesh,
        out_type=jax.ShapeDtypeStruct((batch, seqlen, D), jnp.float32),
        scratch_types=(
            [pltpu.VMEM((b_per_w,), jnp.int32)] * 2
            + [pltpu.VMEM((CHUNK, D), jnp.float32)
               for _ in range(NW_SLOT + NP_SLOT)]
            + [pltpu.SemaphoreType.DMA
               for _ in range(2 * NW_SLOT + NP_SLOT)]
        ),
    )
    def k(ids_hbm, pids_hbm, wtab_hbm, ptab_hbm, out_hbm,
          idx_v, pidx_v, *bufs_sems):
        wbufs = bufs_sems[0:NW_SLOT]
        pbufs = bufs_sems[NW_SLOT:NW_SLOT + NP_SLOT]
        rest = bufs_sems[NW_SLOT + NP_SLOT:]
        sems_w = rest[0:NW_SLOT]
        sems_o = rest[NW_SLOT:2 * NW_SLOT]
        sems_p = rest[2 * NW_SLOT:2 * NW_SLOT + NP_SLOT]

        wid = lax.axis_index("s") * NC + lax.axis_index("c")
        b = wid // w_per_batch
        col0 = (wid % w_per_batch) * b_per_w
        hi = pltpu.async_copy(ids_hbm.at[b, pl.ds(col0, b_per_w)], idx_v,
                              sems_w[0])
        hpi = pltpu.async_copy(pids_hbm.at[b, pl.ds(col0, b_per_w)], pidx_v,
                               sems_p[0])
        hi.wait()
        hpi.wait()

        def start_in(c):
            hw = pltpu.async_copy(
                wtab_hbm.at[idx_v.at[pl.ds(c * CHUNK, CHUNK)]],
                wbufs[c % NW_SLOT], sems_w[c % NW_SLOT])
            hp = pltpu.async_copy(
                ptab_hbm.at[pidx_v.at[pl.ds(c * CHUNK, CHUNK)]],
                pbufs[c % NP_SLOT], sems_p[c % NP_SLOT])
            return hw, hp

        def add_chunk(c):
            wbuf, pbuf = wbufs[c % NW_SLOT], pbufs[c % NP_SLOT]

            def row_body(r, _):
                for j in range(D // LANES):
                    sl = pl.ds(j * LANES, LANES)
                    wbuf[r, sl] = wbuf[r, sl] + pbuf[r, sl]
                return 0

            lax.fori_loop(0, CHUNK, row_body, 0)

        in_flight = {}
        out_flight = {}
        for c in range(min(2, n_chunks)):
            in_flight[c] = start_in(c)

        for c in range(n_chunks):
            hw, hp = in_flight.pop(c)
            hw.wait()
            hp.wait()
            add_chunk(c)
            if (c - 2) in out_flight:
                out_flight.pop(c - 2).wait()
            out_flight[c] = pltpu.async_copy(
                wbufs[c % NW_SLOT],
                out_hbm.at[b, pl.ds(col0 + c * CHUNK, CHUNK)],
                sems_o[c % NW_SLOT])
            if c + 2 < n_chunks:
                in_flight[c + 2] = start_in(c + 2)
        for h in out_flight.values():
            h.wait()

    return k


def kernel(input_ids, position_ids, word_embeddings, position_embeddings):
    batch, seqlen = input_ids.shape
    embed = _make_embed(batch, seqlen)
    return embed(input_ids.astype(jnp.int32), position_ids.astype(jnp.int32),
                 word_embeddings, position_embeddings)
